# Initial kernel scaffold; baseline (speedup 1.0000x reference)
import jax
import jax.numpy as jnp
from jax.experimental import pallas as pl


def kernel(query_tokens, document_tokens, table, mW1, mb1, mW2, mb2, gW1, gb1, gW2, gb2):
    raise NotImplementedError("write your pallas kernel here")


# speedup vs baseline: 6.6204x; 6.6204x over previous
import functools

import jax
import jax.numpy as jnp
from jax import lax
from jax.experimental import pallas as pl
from jax.experimental.pallas import tpu as pltpu
from jax.experimental.pallas import tpu_sc as plsc

V = 1000000
E = 64
BINS = 30
B = 4096
Q = 20
D = 200

N_ROWS = B * (Q + D)
NW = 32
ROWS_PER_W = N_ROWS // NW
GCHUNK = 1408
N_CHUNKS = ROWS_PER_W // GCHUNK

BB = 8
NEG_BIG = -1e30


def _sc_gather_body(idx_hbm, table_hbm, out_hbm, idx_v, rows_v, sem):
    wid = lax.axis_index("s") * 2 + lax.axis_index("c")
    base = wid * ROWS_PER_W

    def chunk(i, carry):
        off = base + i * GCHUNK
        pltpu.sync_copy(idx_hbm.at[pl.ds(off, GCHUNK)], idx_v)
        pltpu.async_copy(table_hbm.at[idx_v], rows_v, sem).wait()
        pltpu.sync_copy(rows_v, out_hbm.at[pl.ds(off, GCHUNK)])
        return carry

    lax.fori_loop(0, N_CHUNKS, chunk, 0, unroll=False)


@jax.jit
def _sc_gather(all_idx, table):
    mesh = plsc.VectorSubcoreMesh(core_axis_name="c", subcore_axis_name="s")
    f = pl.kernel(
        _sc_gather_body,
        out_type=jax.ShapeDtypeStruct((N_ROWS, E), jnp.float32),
        mesh=mesh,
        compiler_params=pltpu.CompilerParams(use_tc_tiling_on_sc=False),
        scratch_types=[
            pltpu.VMEM((GCHUNK,), jnp.int32),
            pltpu.VMEM((GCHUNK, E), jnp.float32),
            pltpu.SemaphoreType.DMA,
        ],
    )
    return f(all_idx, table)


def _tc_body(qe_ref, de_ref, qt_ref, dt_ref, mW1_ref, mb1_ref, mW2_ref,
             mb2_ref, gW1_ref, gb1_ref, gW2_ref, gb2_ref, out_ref, s_ref):
    qm = (qt_ref[...] > 1).astype(jnp.float32)
    dm = (dt_ref[...] > 1).astype(jnp.float32)
    qe = qe_ref[...] * qm
    de = de_ref[...] * dm

    qnorm = jnp.sqrt(jnp.sum(qe * qe, axis=1, keepdims=True))
    dnorm = jnp.sqrt(jnp.sum(de * de, axis=1, keepdims=True))
    qn = qe / jnp.maximum(qnorm, 1e-13)
    dn = de / jnp.maximum(dnorm, 1e-13)

    for i in range(BB):
        qni = qn[i * Q:(i + 1) * Q, :]
        dni = dn[i * D:(i + 1) * D, :]
        s = lax.dot_general(qni, dni, (((1,), (1,)), ((), ())),
                            preferred_element_type=jnp.float32,
                            precision=lax.Precision.HIGHEST)
        s_ref[i * Q:(i + 1) * Q, :] = (s + 1.0) * (BINS / 2.0)

    sv = s_ref[...]
    counts = [jnp.sum((sv >= float(k)).astype(jnp.float32), axis=1,
                      keepdims=True) for k in range(1, BINS)]
    cols = [float(D) - counts[0]]
    cols += [counts[k - 1] - counts[k] for k in range(1, BINS - 1)]
    cols += [counts[BINS - 2]]
    cols += [jnp.zeros_like(cols[0]), jnp.zeros_like(cols[0])]
    hist = jnp.concatenate(cols, axis=1)

    h = jnp.log1p(hist)
    m1 = jnp.tanh(
        lax.dot_general(h, mW1_ref[...], (((1,), (0,)), ((), ())),
                        preferred_element_type=jnp.float32,
                        precision=lax.Precision.HIGHEST) + mb1_ref[...])
    cls = jnp.tanh(
        lax.dot_general(m1, mW2_ref[...], (((1,), (0,)), ((), ())),
                        preferred_element_type=jnp.float32,
                        precision=lax.Precision.HIGHEST)
        + mb2_ref[...])[:, 0:1]

    g1 = jnp.tanh(
        lax.dot_general(qe, gW1_ref[...], (((1,), (0,)), ((), ())),
                        preferred_element_type=jnp.float32,
                        precision=lax.Precision.HIGHEST) + gb1_ref[...])
    graw = jnp.tanh(
        lax.dot_general(g1, gW2_ref[...], (((1,), (0,)), ((), ())),
                        preferred_element_type=jnp.float32,
                        precision=lax.Precision.HIGHEST)
        + gb2_ref[...])[:, 0:1]

    for i in range(BB):
        gr = graw[i * Q:(i + 1) * Q, :]
        qmi = qm[i * Q:(i + 1) * Q, :]
        xm = jnp.where(qmi > 0.0, gr, NEG_BIG)
        xmax = jnp.max(xm, axis=0, keepdims=True)
        ex = jnp.exp(gr - xmax) * qmi
        gate = ex / jnp.sum(ex, axis=0, keepdims=True)
        ci = cls[i * Q:(i + 1) * Q, :]
        out_ref[i:i + 1, :] = jnp.sum(ci * gate, axis=0, keepdims=True)


@jax.jit
def _tc_compute(qe2, de2, qt2, dt2, mW1p, mb1p, mW2p, mb2p, gW1, gb1p,
                gW2p, gb2p):
    nsteps = B // BB

    def wspec(r, c):
        return pl.BlockSpec((r, c), lambda i: (0, 0))

    return pl.pallas_call(
        _tc_body,
        grid=(nsteps,),
        in_specs=[
            pl.BlockSpec((BB * Q, E), lambda i: (i, 0)),
            pl.BlockSpec((BB * D, E), lambda i: (i, 0)),
            pl.BlockSpec((BB * Q, 1), lambda i: (i, 0)),
            pl.BlockSpec((BB * D, 1), lambda i: (i, 0)),
            wspec(32, 32), wspec(1, 32), wspec(32, 128), wspec(1, 128),
            wspec(E, E), wspec(1, E), wspec(E, 128), wspec(1, 128),
        ],
        out_specs=pl.BlockSpec((BB, 1), lambda i: (i, 0)),
        out_shape=jax.ShapeDtypeStruct((B, 1), jnp.float32),
        scratch_shapes=[pltpu.VMEM((BB * Q, D), jnp.float32)],
    )(qe2, de2, qt2, dt2, mW1p, mb1p, mW2p, mb2p, gW1, gb1p, gW2p, gb2p)


def kernel(query_tokens, document_tokens, table, mW1, mb1, mW2, mb2,
           gW1, gb1, gW2, gb2):
    all_idx = jnp.concatenate(
        [query_tokens.reshape(-1), document_tokens.reshape(-1)])
    emb = _sc_gather(all_idx, table)
    qe2 = emb[:B * Q]
    de2 = emb[B * Q:]
    qt2 = query_tokens.reshape(B * Q, 1)
    dt2 = document_tokens.reshape(B * D, 1)

    mW1p = jnp.zeros((32, 32), jnp.float32).at[:BINS, :BINS].set(mW1)
    mb1p = jnp.zeros((1, 32), jnp.float32).at[0, :BINS].set(mb1)
    mW2p = jnp.zeros((32, 128), jnp.float32).at[:BINS, 0].set(mW2[:, 0])
    mb2p = jnp.full((1, 128), mb2[0], jnp.float32)
    gb1p = gb1.reshape(1, E)
    gW2p = jnp.zeros((E, 128), jnp.float32).at[:, 0].set(gW2[:, 0])
    gb2p = jnp.full((1, 128), gb2[0], jnp.float32)

    return _tc_compute(qe2, de2, qt2, dt2, mW1p, mb1p, mW2p, mb2p, gW1,
                       gb1p, gW2p, gb2p)


# speedup vs baseline: 8.6120x; 1.3008x over previous
import jax
import jax.numpy as jnp
from jax import lax
from jax.experimental import pallas as pl
from jax.experimental.pallas import tpu as pltpu
from jax.experimental.pallas import tpu_sc as plsc

V = 1000000
E = 64
BINS = 30
B = 4096
Q = 20
QP = 24
D = 200
DP = 256

N_ROWS = B * (QP + D)
NW = 32
ROWS_PER_W = N_ROWS // NW
GCHUNK = 1792
N_CHUNKS = ROWS_PER_W // GCHUNK

BB = 8
NEG_BIG = -1e30


def _sc_gather_body(idx_hbm, table_hbm, out_hbm, idx_v, rows_v, sem):
    wid = lax.axis_index("s") * 2 + lax.axis_index("c")
    base = wid * ROWS_PER_W

    def chunk(i, carry):
        off = base + i * GCHUNK
        pltpu.sync_copy(idx_hbm.at[pl.ds(off, GCHUNK)], idx_v)
        pltpu.async_copy(table_hbm.at[idx_v], rows_v, sem).wait()
        pltpu.sync_copy(rows_v, out_hbm.at[pl.ds(off, GCHUNK)])
        return carry

    lax.fori_loop(0, N_CHUNKS, chunk, 0, unroll=False)


@jax.jit
def _sc_gather(all_idx, table):
    mesh = plsc.VectorSubcoreMesh(core_axis_name="c", subcore_axis_name="s")
    f = pl.kernel(
        _sc_gather_body,
        out_type=jax.ShapeDtypeStruct((N_ROWS, E), jnp.float32),
        mesh=mesh,
        compiler_params=pltpu.CompilerParams(use_tc_tiling_on_sc=False),
        scratch_types=[
            pltpu.VMEM((GCHUNK,), jnp.int32),
            pltpu.VMEM((GCHUNK, E), jnp.float32),
            pltpu.SemaphoreType.DMA,
        ],
    )
    return f(all_idx, table)


def _tc_body(qe_ref, de_ref, qt_ref, dtr_ref, w_ref, mW1_ref, mb1_ref,
             mW2_ref, mb2_ref, gW1_ref, gb1_ref, gW2_ref, gb2_ref, out_ref,
             s_ref):
    qm = (qt_ref[...] > 1).astype(jnp.float32)
    dmr = (dtr_ref[0] > 1).astype(jnp.float32)
    qe = qe_ref[...]
    de = de_ref[...]

    ones_e = jnp.ones((E, 8), jnp.float32)
    ones_1e = jnp.ones((1, E), jnp.float32)
    qnorm2 = lax.dot_general(qe * qe, ones_e, (((1,), (0,)), ((), ())),
                             preferred_element_type=jnp.float32)[:, 0:1]
    rq = qm * (1.0 / jnp.maximum(jnp.sqrt(qnorm2), 1e-13))
    qn = qe * lax.dot_general(rq, ones_1e, (((1,), (0,)), ((), ())),
                              preferred_element_type=jnp.float32)

    ones_8e = jnp.ones((8, E), jnp.float32)
    dnorm2r = lax.dot_general(ones_8e, de * de, (((1,), (1,)), ((), ())),
                              preferred_element_type=jnp.float32)[0:1, :]
    rdr = dmr * (1.0 / jnp.maximum(jnp.sqrt(dnorm2r), 1e-13))

    for i in range(BB):
        qni = qn[i * QP:(i + 1) * QP, :]
        dei = de[i * D:(i + 1) * D, :]
        raw = lax.dot_general(qni, dei, (((1,), (1,)), ((), ())),
                              preferred_element_type=jnp.float32)
        cos = raw * rdr[:, i * D:(i + 1) * D]
        s_ref[i * QP:(i + 1) * QP, 0:D] = (cos + 1.0) * (BINS / 2.0)
    s_ref[:, D:DP] = jnp.full((BB * QP, DP - D), -1.0, jnp.float32)

    sv = s_ref[...]
    terms = [lax.dot_general((sv >= float(k)).astype(jnp.bfloat16),
                             w_ref[(k - 1) * DP:k * DP, :],
                             (((1,), (0,)), ((), ())),
                             preferred_element_type=jnp.float32)
             for k in range(1, BINS)]
    while len(terms) > 1:
        terms = [terms[i] + terms[i + 1] for i in range(0, len(terms) - 1, 2)] \
            + ([terms[-1]] if len(terms) % 2 else [])
    lane = lax.broadcasted_iota(jnp.int32, (1, 32), 1)
    hist = terms[0] + jnp.where(lane == 0, float(D), 0.0)

    h = jnp.log1p(hist)
    m1 = jnp.tanh(
        lax.dot_general(h, mW1_ref[...], (((1,), (0,)), ((), ())),
                        preferred_element_type=jnp.float32) + mb1_ref[...])
    cls = jnp.tanh(
        lax.dot_general(m1, mW2_ref[...], (((1,), (0,)), ((), ())),
                        preferred_element_type=jnp.float32)[:, 0:1]
        + mb2_ref[...])

    g1 = jnp.tanh(
        qm * lax.dot_general(qe, gW1_ref[...], (((1,), (0,)), ((), ())),
                             preferred_element_type=jnp.float32)
        + gb1_ref[...])
    graw = jnp.tanh(
        lax.dot_general(g1, gW2_ref[...], (((1,), (0,)), ((), ())),
                        preferred_element_type=jnp.float32)[:, 0:1]
        + gb2_ref[...])

    for i in range(BB):
        gr = graw[i * QP:(i + 1) * QP, :]
        qmi = qm[i * QP:(i + 1) * QP, :]
        xm = jnp.where(qmi > 0.0, gr, NEG_BIG)
        xmax = jnp.max(xm, axis=0, keepdims=True)
        ex = jnp.exp(gr - xmax) * qmi
        gate = ex / jnp.sum(ex, axis=0, keepdims=True)
        ci = cls[i * QP:(i + 1) * QP, :]
        out_ref[i:i + 1, :] = jnp.sum(ci * gate, axis=0, keepdims=True)


@jax.jit
def _tc_compute(qe2, de2, qt2, dtr, wsgn, mW1p, mb1p, mW2p, mb2p, gW1,
                gb1p, gW2p, gb2p):
    nsteps = B // BB

    def wspec(r, c):
        return pl.BlockSpec((r, c), lambda i: (0, 0))

    return pl.pallas_call(
        _tc_body,
        grid=(nsteps,),
        in_specs=[
            pl.BlockSpec((BB * QP, E), lambda i: (i, 0)),
            pl.BlockSpec((BB * D, E), lambda i: (i, 0)),
            pl.BlockSpec((BB * QP, 1), lambda i: (i, 0)),
            pl.BlockSpec((1, 1, BB * D), lambda i: (i, 0, 0)),
            wspec((BINS - 1) * DP, 32),
            wspec(32, 32), wspec(1, 32), wspec(32, 8), wspec(1, 1),
            wspec(E, E), wspec(1, E), wspec(E, 8), wspec(1, 1),
        ],
        out_specs=pl.BlockSpec((BB, 1), lambda i: (i, 0)),
        out_shape=jax.ShapeDtypeStruct((B, 1), jnp.float32),
        scratch_shapes=[pltpu.VMEM((BB * QP, DP), jnp.float32)],
    )(qe2, de2, qt2, dtr, wsgn, mW1p, mb1p, mW2p, mb2p, gW1, gb1p, gW2p,
      gb2p)


def kernel(query_tokens, document_tokens, table, mW1, mb1, mW2, mb2,
           gW1, gb1, gW2, gb2):
    qtp = jnp.pad(query_tokens, ((0, 0), (0, QP - Q)))
    pad_rows = (jnp.arange(B * (QP - Q), dtype=jnp.int32) % V).reshape(
        B, QP - Q)
    gidx = jnp.concatenate([query_tokens, pad_rows], axis=1)
    all_idx = jnp.concatenate(
        [gidx.reshape(-1), document_tokens.reshape(-1)])
    emb = _sc_gather(all_idx, table)
    qe2 = emb[:B * QP]
    de2 = emb[B * QP:]
    qt2 = qtp.reshape(B * QP, 1)
    dtr = document_tokens.reshape(B // BB, 1, BB * D)

    kk = jnp.arange(1, BINS)[:, None, None]
    lane32 = jnp.arange(32)[None, None, :]
    wsgn = jnp.where(lane32 == kk, 1.0,
                     jnp.where(lane32 == kk - 1, -1.0, 0.0))
    wsgn = jnp.broadcast_to(wsgn, (BINS - 1, DP, 32)).reshape(
        (BINS - 1) * DP, 32).astype(jnp.bfloat16)

    mW1p = jnp.zeros((32, 32), jnp.float32).at[:BINS, :BINS].set(mW1)
    mb1p = jnp.zeros((1, 32), jnp.float32).at[0, :BINS].set(mb1)
    mW2p = jnp.zeros((32, 8), jnp.float32).at[:BINS, 0].set(mW2[:, 0])
    mb2p = mb2.reshape(1, 1)
    gb1p = gb1.reshape(1, E)
    gW2p = jnp.zeros((E, 8), jnp.float32).at[:, 0].set(gW2[:, 0])
    gb2p = gb2.reshape(1, 1)

    return _tc_compute(qe2, de2, qt2, dtr, wsgn, mW1p, mb1p, mW2p, mb2p,
                       gW1, gb1p, gW2p, gb2p)
